# Initial kernel scaffold; baseline (speedup 1.0000x reference)
import functools

import jax
import jax.numpy as jnp
from jax import lax
from jax.experimental import pallas as pl
from jax.experimental.pallas import tpu as pltpu
from jax.experimental.pallas import tpu_sc as plsc

NC = 2
NS = 16
CH = 80


def _segsum_body(with_counts, n_nodes, n_feat, e_per_w, n_chunks, rows_per_tile,
                 *refs):
  if with_counts:
    (x_hbm, src_hbm, dst_hbm, zrow_hbm, zcnt_hbm, ones_hbm,
     p_out, c_out,
     src_i, dst_i, rows_v, ones_v, acc, cnt, gsem) = refs
  else:
    (x_hbm, src_hbm, dst_hbm, zrow_hbm,
     p_out,
     src_i, dst_i, rows_v, acc, gsem) = refs

  c = lax.axis_index("c")
  s = lax.axis_index("s")
  wid = s * NC + c

  row_base = s * rows_per_tile
  pltpu.sync_copy(zrow_hbm, acc.at[pl.ds(row_base, rows_per_tile)])
  if with_counts:
    pltpu.sync_copy(zcnt_hbm, cnt.at[pl.ds(row_base, rows_per_tile)])
    pltpu.sync_copy(ones_hbm, ones_v)
  plsc.subcore_barrier()

  edge_base = wid * e_per_w

  def body(j, carry):
    base = edge_base + j * CH
    pltpu.sync_copy(src_hbm.at[pl.ds(base, CH)], src_i)
    pltpu.sync_copy(dst_hbm.at[pl.ds(base, CH)], dst_i)
    pltpu.async_copy(x_hbm.at[src_i], rows_v, gsem).wait()
    pltpu.sync_copy(rows_v, acc.at[dst_i], add=True)
    if with_counts:
      pltpu.sync_copy(ones_v, cnt.at[dst_i], add=True)
    return carry

  lax.fori_loop(0, n_chunks, body, 0)
  plsc.subcore_barrier()

  pltpu.sync_copy(acc.at[pl.ds(row_base, rows_per_tile)],
                  p_out.at[c, pl.ds(row_base, rows_per_tile)])
  if with_counts:
    pltpu.sync_copy(cnt.at[pl.ds(row_base, rows_per_tile)],
                    c_out.at[c, pl.ds(row_base, rows_per_tile)])


@functools.partial(jax.jit, static_argnums=(3,))
def _segsum(x, src, dst, with_counts):
  n_nodes, n_feat = x.shape
  n_edges = src.shape[0]
  nw = NC * NS
  e_per_w = n_edges // nw
  n_chunks = e_per_w // CH
  assert e_per_w * nw == n_edges and n_chunks * CH == e_per_w
  rows_per_tile = n_nodes // NS
  assert rows_per_tile * NS == n_nodes

  mesh = plsc.VectorSubcoreMesh(core_axis_name="c", subcore_axis_name="s")
  out_type = [jax.ShapeDtypeStruct((NC, n_nodes, n_feat), jnp.float32)]
  scratch = [
      pltpu.VMEM((CH,), jnp.int32),
      pltpu.VMEM((CH,), jnp.int32),
      pltpu.VMEM((CH, n_feat), jnp.float32),
  ]
  inputs = [x, src, dst, jnp.zeros((rows_per_tile, n_feat), jnp.float32)]
  if with_counts:
    out_type.append(jax.ShapeDtypeStruct((NC, n_nodes, 16), jnp.float32))
    scratch.append(pltpu.VMEM((CH, 16), jnp.float32))
    inputs.append(jnp.zeros((rows_per_tile, 16), jnp.float32))
    inputs.append(jnp.ones((CH, 16), jnp.float32))
  scratch.append(pltpu.VMEM_SHARED((n_nodes, n_feat), jnp.float32))
  if with_counts:
    scratch.append(pltpu.VMEM_SHARED((n_nodes, 16), jnp.float32))
  scratch.append(pltpu.SemaphoreType.DMA)

  body = functools.partial(_segsum_body, with_counts, n_nodes, n_feat,
                           e_per_w, n_chunks, rows_per_tile)
  fn = pl.kernel(body, out_type=out_type, mesh=mesh, scratch_types=scratch)
  return fn(*inputs)


def _dense_body(relu, p_ref, c_ref, x_ref, wl_ref, b_ref, wr_ref, o_ref):
  ssum = p_ref[0] + p_ref[1]
  cnt = c_ref[0][:, 0:1] + c_ref[1][:, 0:1]
  mean = ssum / jnp.maximum(cnt, 1.0)
  acc = lax.dot_general(mean, wl_ref[...], (((1,), (1,)), ((), ())),
                        preferred_element_type=jnp.float32)
  acc = acc + lax.dot_general(x_ref[...], wr_ref[...], (((1,), (1,)), ((), ())),
                              preferred_element_type=jnp.float32)
  acc = acc + b_ref[...]
  o_ref[...] = jnp.maximum(acc, 0.0) if relu else acc


def _dense(p, cpart, x, wl, bl, wr, relu):
  n, f = x.shape
  blk = 1000
  grid = (n // blk,)
  body = functools.partial(_dense_body, relu)
  return pl.pallas_call(
      body,
      grid=grid,
      in_specs=[
          pl.BlockSpec((NC, blk, f), lambda i: (0, i, 0)),
          pl.BlockSpec((NC, blk, 16), lambda i: (0, i, 0)),
          pl.BlockSpec((blk, f), lambda i: (i, 0)),
          pl.BlockSpec((f, f), lambda i: (0, 0)),
          pl.BlockSpec((1, f), lambda i: (0, 0)),
          pl.BlockSpec((f, f), lambda i: (0, 0)),
      ],
      out_specs=pl.BlockSpec((blk, f), lambda i: (i, 0)),
      out_shape=jax.ShapeDtypeStruct((n, f), jnp.float32),
  )(p, cpart, x, wl, bl.reshape(1, f), wr)


def kernel(x, edge_index, lin1_W, lin1_b, c1_Wl, c1_bl, c1_Wr,
           c2_Wl, c2_bl, c2_Wr):
  e32 = edge_index.astype(jnp.int32)
  src = e32[0]
  dst = e32[1]

  p1, cpart = _segsum(x, src, dst, True)
  h = _dense(p1, cpart, x, c1_Wl, c1_bl, c1_Wr, True)
  (p2,) = _segsum(h, src, dst, False)
  out = _dense(p2, cpart, h, c2_Wl, c2_bl, c2_Wr, False)
  return out


# speedup vs baseline: 3.9177x; 3.9177x over previous
import functools

import jax
import jax.numpy as jnp
from jax import lax
from jax.experimental import pallas as pl
from jax.experimental.pallas import tpu as pltpu
from jax.experimental.pallas import tpu_sc as plsc

NC = 2
NS = 16
CH = 80


RB = 16


def _segsum_body(with_counts, n_nodes, n_feat, e_per_w, n_chunks,
                 *refs):
  if with_counts:
    (x_hbm, src_hbm, dst_hbm, zrow_hbm, ones_hbm,
     p_out, c_out,
     src_i, dst_i, rows_v, ones_v, acc, gsem) = refs
  else:
    (x_hbm, src_hbm, dst_hbm, zrow_hbm,
     p_out,
     src_i, dst_i, rows_v, acc, gsem) = refs

  c = lax.axis_index("c")
  s = lax.axis_index("s")
  wid = s * NC + c

  n_rb = n_nodes // RB
  rb_per_tile = (n_rb + NS - 1) // NS
  edge_base = wid * e_per_w

  def zero_acc():
    def zero_body(k, carry):
      cid = s * rb_per_tile + k

      @pl.when(cid < n_rb)
      def _():
        pltpu.sync_copy(zrow_hbm, acc.at[pl.ds(cid * RB, RB)])
      return carry

    lax.fori_loop(0, rb_per_tile, zero_body, 0)

  def write_out(dst_hbm_arr):
    def out_body(k, carry):
      cid = s * rb_per_tile + k

      @pl.when(cid < n_rb)
      def _():
        pltpu.sync_copy(acc.at[pl.ds(cid * RB, RB)],
                        dst_hbm_arr.at[c, pl.ds(cid * RB, RB)])
      return carry

    lax.fori_loop(0, rb_per_tile, out_body, 0)

  if with_counts:
    zero_acc()
    pltpu.sync_copy(ones_hbm, ones_v)
    plsc.subcore_barrier()

    def cbody(j, carry):
      base = edge_base + j * CH
      pltpu.sync_copy(dst_hbm.at[pl.ds(base, CH)], dst_i)
      pltpu.sync_copy(ones_v, acc.at[dst_i], add=True)
      return carry

    lax.fori_loop(0, n_chunks, cbody, 0)
    plsc.subcore_barrier()
    write_out(c_out)
    plsc.subcore_barrier()

  zero_acc()
  plsc.subcore_barrier()

  def body(j, carry):
    base = edge_base + j * CH
    pltpu.sync_copy(src_hbm.at[pl.ds(base, CH)], src_i)
    pltpu.sync_copy(dst_hbm.at[pl.ds(base, CH)], dst_i)
    pltpu.async_copy(x_hbm.at[src_i], rows_v, gsem).wait()
    pltpu.sync_copy(rows_v, acc.at[dst_i], add=True)
    return carry

  lax.fori_loop(0, n_chunks, body, 0)
  plsc.subcore_barrier()
  write_out(p_out)


@functools.partial(jax.jit, static_argnums=(3,))
def _segsum(x, src, dst, with_counts):
  n_nodes, n_feat = x.shape
  n_edges = src.shape[0]
  nw = NC * NS
  e_per_w = n_edges // nw
  n_chunks = e_per_w // CH
  assert e_per_w * nw == n_edges and n_chunks * CH == e_per_w
  assert n_nodes % RB == 0

  mesh = plsc.VectorSubcoreMesh(core_axis_name="c", subcore_axis_name="s")
  out_type = [jax.ShapeDtypeStruct((NC, n_nodes, n_feat), jnp.float32)]
  scratch = [
      pltpu.VMEM((CH,), jnp.int32),
      pltpu.VMEM((CH,), jnp.int32),
      pltpu.VMEM((CH, n_feat), jnp.float32),
  ]
  inputs = [x, src, dst, jnp.zeros((RB, n_feat), jnp.float32)]
  if with_counts:
    out_type.append(jax.ShapeDtypeStruct((NC, n_nodes, n_feat), jnp.float32))
    scratch.append(pltpu.VMEM((CH, n_feat), jnp.float32))
    inputs.append(jnp.ones((CH, n_feat), jnp.float32))
  scratch.append(pltpu.VMEM_SHARED((n_nodes, n_feat), jnp.float32))
  scratch.append(pltpu.SemaphoreType.DMA)

  body = functools.partial(_segsum_body, with_counts, n_nodes, n_feat,
                           e_per_w, n_chunks)
  fn = pl.kernel(body, out_type=out_type, mesh=mesh, scratch_types=scratch)
  return fn(*inputs)


def _dense_body(relu, p_ref, c_ref, x_ref, wl_ref, b_ref, wr_ref, o_ref):
  ssum = p_ref[0] + p_ref[1]
  cnt = c_ref[0][:, 0:1] + c_ref[1][:, 0:1]
  mean = ssum / jnp.maximum(cnt, 1.0)
  acc = lax.dot_general(mean, wl_ref[...], (((1,), (1,)), ((), ())),
                        preferred_element_type=jnp.float32)
  acc = acc + lax.dot_general(x_ref[...], wr_ref[...], (((1,), (1,)), ((), ())),
                              preferred_element_type=jnp.float32)
  acc = acc + b_ref[...]
  o_ref[...] = jnp.maximum(acc, 0.0) if relu else acc


def _dense(p, cpart, x, wl, bl, wr, relu):
  n, f = x.shape
  blk = 1000
  grid = (n // blk,)
  body = functools.partial(_dense_body, relu)
  return pl.pallas_call(
      body,
      grid=grid,
      in_specs=[
          pl.BlockSpec((NC, blk, f), lambda i: (0, i, 0)),
          pl.BlockSpec((NC, blk, f), lambda i: (0, i, 0)),
          pl.BlockSpec((blk, f), lambda i: (i, 0)),
          pl.BlockSpec((f, f), lambda i: (0, 0)),
          pl.BlockSpec((1, f), lambda i: (0, 0)),
          pl.BlockSpec((f, f), lambda i: (0, 0)),
      ],
      out_specs=pl.BlockSpec((blk, f), lambda i: (i, 0)),
      out_shape=jax.ShapeDtypeStruct((n, f), jnp.float32),
  )(p, cpart, x, wl, bl.reshape(1, f), wr)


def kernel(x, edge_index, lin1_W, lin1_b, c1_Wl, c1_bl, c1_Wr,
           c2_Wl, c2_bl, c2_Wr):
  e32 = edge_index.astype(jnp.int32)
  src = e32[0]
  dst = e32[1]

  p1, cpart = _segsum(x, src, dst, True)
  h = _dense(p1, cpart, x, c1_Wl, c1_bl, c1_Wr, True)
  (p2,) = _segsum(h, src, dst, False)
  out = _dense(p2, cpart, h, c2_Wl, c2_bl, c2_Wr, False)
  return out


# speedup vs baseline: 10.4098x; 2.6571x over previous
import functools

import jax
import jax.numpy as jnp
from jax import lax
from jax.experimental import pallas as pl
from jax.experimental.pallas import tpu as pltpu
from jax.experimental.pallas import tpu_sc as plsc

NC = 2
NS = 16
CH = 80


def _segsum_body(with_counts, n_nodes, n_feat, e_per_w, n_chunks, big,
                 *refs):
  if with_counts:
    (x_hbm, src_hbm, dst_hbm, zrow_hbm, ones_hbm,
     p_out, c_out,
     src_all, dst_all, rows0, rows1, ones_v, acc, gsem0, gsem1) = refs
  else:
    (x_hbm, src_hbm, dst_hbm, zrow_hbm,
     p_out,
     src_all, dst_all, rows0, rows1, acc, gsem0, gsem1) = refs

  c = lax.axis_index("c")
  s = lax.axis_index("s")
  wid = s * NC + c
  tail = n_nodes - NS * big

  def zero_acc():
    pltpu.sync_copy(zrow_hbm, acc.at[pl.ds(s * big, big)])

    @pl.when(s == 0)
    def _():
      pltpu.sync_copy(zrow_hbm.at[pl.ds(0, tail)],
                      acc.at[pl.ds(NS * big, tail)])

  def write_out(dst_hbm_arr):
    pltpu.sync_copy(acc.at[pl.ds(s * big, big)],
                    dst_hbm_arr.at[c, pl.ds(s * big, big)])

    @pl.when(s == 0)
    def _():
      pltpu.sync_copy(acc.at[pl.ds(NS * big, tail)],
                      dst_hbm_arr.at[c, pl.ds(NS * big, tail)])

  ebase = wid * e_per_w
  pltpu.sync_copy(src_hbm.at[pl.ds(ebase, e_per_w)], src_all)
  pltpu.sync_copy(dst_hbm.at[pl.ds(ebase, e_per_w)], dst_all)

  if with_counts:
    zero_acc()
    pltpu.sync_copy(ones_hbm, ones_v)
    plsc.subcore_barrier()

    @pl.loop(0, n_chunks)
    def _(j):
      pltpu.sync_copy(ones_v, acc.at[dst_all.at[pl.ds(j * CH, CH)]], add=True)

    plsc.subcore_barrier()
    write_out(c_out)
    plsc.subcore_barrier()

  zero_acc()
  plsc.subcore_barrier()

  rows = (rows0, rows1)
  gsem = (gsem0, gsem1)

  def gather_start(j, b):
    pltpu.make_async_copy(
        x_hbm.at[src_all.at[pl.ds(j * CH, CH)]], rows[b], gsem[b]).start()

  def gather_wait(j, b):
    pltpu.make_async_copy(
        x_hbm.at[src_all.at[pl.ds(j * CH, CH)]], rows[b], gsem[b]).wait()

  def scatter(j, b):
    pltpu.sync_copy(rows[b], acc.at[dst_all.at[pl.ds(j * CH, CH)]], add=True)

  gather_start(0, 0)

  @pl.loop(0, n_chunks - 1, step=2)
  def _(j0):
    for b in (0, 1):
      j = j0 + b
      gather_start(j + 1, 1 - b)
      gather_wait(j, b)
      scatter(j, b)

  gather_wait(n_chunks - 1, 0)
  scatter(n_chunks - 1, 0)
  plsc.subcore_barrier()
  write_out(p_out)


@functools.partial(jax.jit, static_argnums=(3,))
def _segsum(x, src, dst, with_counts):
  n_nodes, n_feat = x.shape
  n_edges = src.shape[0]
  nw = NC * NS
  e_per_w = n_edges // nw
  n_chunks = e_per_w // CH
  assert e_per_w * nw == n_edges and n_chunks * CH == e_per_w
  assert n_chunks % 2 == 1
  big = (n_nodes // NS) // 8 * 8
  assert 0 < n_nodes - NS * big <= big

  mesh = plsc.VectorSubcoreMesh(core_axis_name="c", subcore_axis_name="s")
  out_type = [jax.ShapeDtypeStruct((NC, n_nodes, n_feat), jnp.float32)]
  scratch = [
      pltpu.VMEM((e_per_w,), jnp.int32),
      pltpu.VMEM((e_per_w,), jnp.int32),
      pltpu.VMEM((CH, n_feat), jnp.float32),
      pltpu.VMEM((CH, n_feat), jnp.float32),
  ]
  inputs = [x, src, dst, jnp.zeros((big, n_feat), jnp.float32)]
  if with_counts:
    out_type.append(jax.ShapeDtypeStruct((NC, n_nodes, n_feat), jnp.float32))
    scratch.append(pltpu.VMEM((CH, n_feat), jnp.float32))
    inputs.append(jnp.ones((CH, n_feat), jnp.float32))
  scratch.append(pltpu.VMEM_SHARED((n_nodes, n_feat), jnp.float32))
  scratch.append(pltpu.SemaphoreType.DMA)
  scratch.append(pltpu.SemaphoreType.DMA)

  body = functools.partial(_segsum_body, with_counts, n_nodes, n_feat,
                           e_per_w, n_chunks, big)
  fn = pl.kernel(body, out_type=out_type, mesh=mesh, scratch_types=scratch)
  return fn(*inputs)


def _dense_body(relu, p_ref, c_ref, x_ref, wl_ref, b_ref, wr_ref, o_ref):
  ssum = p_ref[0] + p_ref[1]
  cnt = c_ref[0][:, 0:1] + c_ref[1][:, 0:1]
  mean = ssum / jnp.maximum(cnt, 1.0)
  acc = lax.dot_general(mean, wl_ref[...], (((1,), (1,)), ((), ())),
                        preferred_element_type=jnp.float32)
  acc = acc + lax.dot_general(x_ref[...], wr_ref[...], (((1,), (1,)), ((), ())),
                              preferred_element_type=jnp.float32)
  acc = acc + b_ref[...]
  o_ref[...] = jnp.maximum(acc, 0.0) if relu else acc


def _dense(p, cpart, x, wl, bl, wr, relu):
  n, f = x.shape
  blk = 1000
  grid = (n // blk,)
  body = functools.partial(_dense_body, relu)
  return pl.pallas_call(
      body,
      grid=grid,
      in_specs=[
          pl.BlockSpec((NC, blk, f), lambda i: (0, i, 0)),
          pl.BlockSpec((NC, blk, f), lambda i: (0, i, 0)),
          pl.BlockSpec((blk, f), lambda i: (i, 0)),
          pl.BlockSpec((f, f), lambda i: (0, 0)),
          pl.BlockSpec((1, f), lambda i: (0, 0)),
          pl.BlockSpec((f, f), lambda i: (0, 0)),
      ],
      out_specs=pl.BlockSpec((blk, f), lambda i: (i, 0)),
      out_shape=jax.ShapeDtypeStruct((n, f), jnp.float32),
  )(p, cpart, x, wl, bl.reshape(1, f), wr)


def kernel(x, edge_index, lin1_W, lin1_b, c1_Wl, c1_bl, c1_Wr,
           c2_Wl, c2_bl, c2_Wr):
  e32 = edge_index.astype(jnp.int32)
  src = e32[0]
  dst = e32[1]

  p1, cpart = _segsum(x, src, dst, True)
  h = _dense(p1, cpart, x, c1_Wl, c1_bl, c1_Wr, True)
  (p2,) = _segsum(h, src, dst, False)
  out = _dense(p2, cpart, h, c2_Wl, c2_bl, c2_Wr, False)
  return out
